# Initial kernel scaffold; baseline (speedup 1.0000x reference)
import jax
import jax.numpy as jnp
from jax.experimental import pallas as pl


def kernel(x, y, lum_w, bin_edges):
    raise NotImplementedError("write your pallas kernel here")


# speedup vs baseline: 1725.7741x; 1725.7741x over previous
import jax
import jax.numpy as jnp
from jax import lax
from jax.experimental import pallas as pl
from jax.experimental.pallas import tpu as pltpu
from jax.experimental.pallas import tpu_sc as plsc

B = 32
NCH = 3
NPIX = 512 * 512
NBINS = 64
LANES = 16
CHUNK = 8192
NSTEPS = NPIX // CHUNK
NVREG = CHUNK // LANES
NC, NS = 2, 16
HSIZE = NBINS * LANES
LAMBDA_CC = 10.0
EPS = 1e-06


def _sc_hist_body(x_hbm, y_hbm, w_hbm, hx_out, hy_out, s_out,
                  xbuf, ybuf, wbuf, hx, hy, sums_v, sem0, sem1):
    img = lax.axis_index("c") * NS + lax.axis_index("s")
    sems = (sem0, sem1)

    pltpu.sync_copy(w_hbm, wbuf)
    w0 = wbuf[pl.ds(0, LANES)]
    w1 = wbuf[pl.ds(LANES, LANES)]
    w2 = wbuf[pl.ds(2 * LANES, LANES)]
    woff = wbuf[pl.ds(3 * LANES, LANES)]
    li = lax.iota(jnp.int32, LANES)
    ones = jnp.full((LANES,), 1.0, jnp.float32)
    zeros = jnp.zeros((LANES,), jnp.float32)

    for i in range(NBINS):
        hx[pl.ds(i * LANES, LANES)] = zeros
        hy[pl.ds(i * LANES, LANES)] = zeros

    def start(s, b):
        for c in range(NCH):
            pltpu.async_copy(
                x_hbm.at[pl.ds((img * NCH + c) * NPIX + s * CHUNK, CHUNK)],
                xbuf.at[pl.ds((b * NCH + c) * CHUNK, CHUNK)], sems[b])
            pltpu.async_copy(
                y_hbm.at[pl.ds((img * NCH + c) * NPIX + s * CHUNK, CHUNK)],
                ybuf.at[pl.ds((b * NCH + c) * CHUNK, CHUNK)], sems[b])

    def wait(b):
        for c in range(NCH):
            pltpu.make_async_copy(
                x_hbm.at[pl.ds(0, CHUNK)],
                xbuf.at[pl.ds((b * NCH + c) * CHUNK, CHUNK)], sems[b]).wait()
            pltpu.make_async_copy(
                y_hbm.at[pl.ds(0, CHUNK)],
                ybuf.at[pl.ds((b * NCH + c) * CHUNK, CHUNK)], sems[b]).wait()

    def chunk_compute(b, carry):
        xo = b * NCH * CHUNK
        yo = b * NCH * CHUNK

        def jbody(j, cr):
            sxr, sxg, sxb, syr, syg, syb = cr
            o = j * LANES
            r = xbuf[pl.ds(xo + o, LANES)]
            g = xbuf[pl.ds(xo + CHUNK + o, LANES)]
            bl = xbuf[pl.ds(xo + 2 * CHUNK + o, LANES)]
            cx = r * w0 + g * w1 + bl * w2 + woff
            ix = jnp.minimum(jnp.maximum(cx, 0.0), 63.0).astype(jnp.int32)
            plsc.addupdate_scatter(hx, [ix * LANES + li], ones)
            r2 = ybuf[pl.ds(yo + o, LANES)]
            g2 = ybuf[pl.ds(yo + CHUNK + o, LANES)]
            bl2 = ybuf[pl.ds(yo + 2 * CHUNK + o, LANES)]
            cy = r2 * w0 + g2 * w1 + bl2 * w2 + woff
            iy = jnp.minimum(jnp.maximum(cy, 0.0), 63.0).astype(jnp.int32)
            plsc.addupdate_scatter(hy, [iy * LANES + li], ones)
            return (sxr + r, sxg + g, sxb + bl,
                    syr + r2, syg + g2, syb + bl2)

        return lax.fori_loop(0, NVREG, jbody, carry)

    start(0, 0)
    init = (zeros,) * 6

    def outer(i, carry):
        s0 = 2 * i
        start(s0 + 1, 1)
        wait(0)
        carry = chunk_compute(0, carry)

        @pl.when(i < NSTEPS // 2 - 1)
        def _():
            start(s0 + 2, 0)

        wait(1)
        return chunk_compute(1, carry)

    sxr, sxg, sxb, syr, syg, syb = lax.fori_loop(
        0, NSTEPS // 2, outer, init)

    for k, v in enumerate((sxr, sxg, sxb, syr, syg, syb, zeros, zeros)):
        sums_v[pl.ds(k * LANES, LANES)] = v

    pltpu.sync_copy(hx, hx_out.at[pl.ds(img * HSIZE, HSIZE)])
    pltpu.sync_copy(hy, hy_out.at[pl.ds(img * HSIZE, HSIZE)])
    pltpu.sync_copy(sums_v, s_out.at[pl.ds(img * 8 * LANES, 8 * LANES)])


def _sc_hist(xf, yf, wvec):
    mesh = plsc.VectorSubcoreMesh(core_axis_name="c", subcore_axis_name="s",
                                  num_cores=NC, num_subcores=NS)
    f = pl.kernel(
        _sc_hist_body,
        out_type=(
            jax.ShapeDtypeStruct((B * HSIZE,), jnp.float32),
            jax.ShapeDtypeStruct((B * HSIZE,), jnp.float32),
            jax.ShapeDtypeStruct((B * 8 * LANES,), jnp.float32),
        ),
        mesh=mesh,
        scratch_types=[
            pltpu.VMEM((2 * NCH * CHUNK,), jnp.float32),
            pltpu.VMEM((2 * NCH * CHUNK,), jnp.float32),
            pltpu.VMEM((4 * LANES,), jnp.float32),
            pltpu.VMEM((HSIZE,), jnp.float32),
            pltpu.VMEM((HSIZE,), jnp.float32),
            pltpu.VMEM((8 * LANES,), jnp.float32),
            pltpu.SemaphoreType.DMA,
            pltpu.SemaphoreType.DMA,
        ],
        compiler_params=pltpu.CompilerParams(needs_layout_passes=False),
        name="cc_hist_sc",
    )
    return f(xf, yf, wvec)


def _tc_final_body(hx_ref, hy_ref, s_ref, out_ref):
    hx = 2.0 * jnp.sum(hx_ref[...], axis=2)
    hy = 2.0 * jnp.sum(hy_ref[...], axis=2)
    s = jnp.sum(s_ref[...], axis=2)
    m01 = (s * (1.0 / NPIX) + 1.0) * 0.5
    xm = m01[:, 0:3]
    ym = m01[:, 3:6]
    rw = xm[:, 0:1]
    gw = xm[:, 1:2]
    bw = xm[:, 2:3]
    grey_world = jnp.mean(jnp.abs(rw - gw) + jnp.abs(gw - bw) + jnp.abs(bw - rw))
    x_ratio = xm / (jnp.sum(xm, axis=1, keepdims=True) + EPS)
    y_ratio = ym / (jnp.sum(ym, axis=1, keepdims=True) + EPS)
    ratio_loss = jnp.mean(jnp.abs(x_ratio - y_ratio))
    log_x = jnp.log(hx)
    kl_pt = jnp.where(hy > 0,
                      hy * (jnp.log(jnp.where(hy > 0, hy, 1.0)) - log_x), 0.0)
    kl_div = jnp.sum(kl_pt) / B
    out_ref[...] = jnp.full((1, 1), LAMBDA_CC * (grey_world + ratio_loss + kl_div),
                            jnp.float32)


def _tc_final(hx, hy, s):
    return pl.pallas_call(
        _tc_final_body,
        out_shape=jax.ShapeDtypeStruct((1, 1), jnp.float32),
    )(hx, hy, s)


def kernel(x, y, lum_w, bin_edges):
    xf = x.reshape(B * NCH * NPIX)
    yf = y.reshape(B * NCH * NPIX)
    lw = lum_w.reshape(NCH)
    nb = bin_edges.shape[0] - 1
    scale = nb / (bin_edges[-1] - bin_edges[0])
    wrows = 0.5 * scale * lw
    woff = scale * (0.5 * jnp.sum(lw) - bin_edges[0])
    wvec = jnp.concatenate(
        [jnp.broadcast_to(wrows[:, None], (NCH, LANES)),
         jnp.broadcast_to(woff[None, None], (1, LANES))], axis=0).reshape(-1)
    hxf, hyf, sf = _sc_hist(xf, yf, wvec)
    out = _tc_final(hxf.reshape(B, NBINS, LANES),
                    hyf.reshape(B, NBINS, LANES),
                    sf.reshape(B, 8, LANES))
    return out.reshape(())


# speedup vs baseline: 4209.2020x; 2.4390x over previous
import jax
import jax.numpy as jnp
from jax import lax
from jax.experimental import pallas as pl
from jax.experimental.pallas import tpu as pltpu
from jax.experimental.pallas import tpu_sc as plsc

B = 32
NCH = 3
NPIX = 512 * 512
NBINS = 64
LANES = 16
CHUNK = 8192
NSTEPS = NPIX // CHUNK
NVREG = CHUNK // LANES
NC, NS = 2, 16
UNROLL = 4
HSIZE = NBINS * LANES
LAMBDA_CC = 10.0
EPS = 1e-06


def _sc_hist_body(x_hbm, y_hbm, w_hbm, hx_out, hy_out, s_out,
                  xbuf, ybuf, wbuf, hx, hy, sums_v, sem0, sem1):
    img = lax.axis_index("c") * NS + lax.axis_index("s")
    sems = (sem0, sem1)

    pltpu.sync_copy(w_hbm, wbuf)
    w0 = wbuf[pl.ds(0, LANES)]
    w1 = wbuf[pl.ds(LANES, LANES)]
    w2 = wbuf[pl.ds(2 * LANES, LANES)]
    woff = wbuf[pl.ds(3 * LANES, LANES)]
    li = lax.iota(jnp.int32, LANES)
    ones = jnp.full((LANES,), 1.0, jnp.float32)
    zeros = jnp.zeros((LANES,), jnp.float32)

    for i in range(NBINS):
        hx[pl.ds(i * LANES, LANES)] = zeros
        hy[pl.ds(i * LANES, LANES)] = zeros

    def start(s, b):
        for c in range(NCH):
            pltpu.async_copy(
                x_hbm.at[pl.ds((img * NCH + c) * NPIX + s * CHUNK, CHUNK)],
                xbuf.at[pl.ds((b * NCH + c) * CHUNK, CHUNK)], sems[b])
            pltpu.async_copy(
                y_hbm.at[pl.ds((img * NCH + c) * NPIX + s * CHUNK, CHUNK)],
                ybuf.at[pl.ds((b * NCH + c) * CHUNK, CHUNK)], sems[b])

    def wait(b):
        for c in range(NCH):
            pltpu.make_async_copy(
                x_hbm.at[pl.ds(0, CHUNK)],
                xbuf.at[pl.ds((b * NCH + c) * CHUNK, CHUNK)], sems[b]).wait()
            pltpu.make_async_copy(
                y_hbm.at[pl.ds(0, CHUNK)],
                ybuf.at[pl.ds((b * NCH + c) * CHUNK, CHUNK)], sems[b]).wait()

    def chunk_compute(b, carry):
        base = b * NCH * CHUNK

        def jbody(j, cr):
            sxr, sxg, sxb, syr, syg, syb = cr
            o = j * LANES
            r = xbuf[pl.ds(base + o, LANES)]
            g = xbuf[pl.ds(base + CHUNK + o, LANES)]
            bl = xbuf[pl.ds(base + 2 * CHUNK + o, LANES)]
            cx = r * w0 + g * w1 + bl * w2 + woff
            ix = jnp.minimum(jnp.maximum(cx, 0.0), 63.0).astype(jnp.int32)
            plsc.addupdate_scatter(hx, [ix * LANES + li], ones)
            r2 = ybuf[pl.ds(base + o, LANES)]
            g2 = ybuf[pl.ds(base + CHUNK + o, LANES)]
            bl2 = ybuf[pl.ds(base + 2 * CHUNK + o, LANES)]
            cy = r2 * w0 + g2 * w1 + bl2 * w2 + woff
            iy = jnp.minimum(jnp.maximum(cy, 0.0), 63.0).astype(jnp.int32)
            plsc.addupdate_scatter(hy, [iy * LANES + li], ones)
            return (sxr + r, sxg + g, sxb + bl,
                    syr + r2, syg + g2, syb + bl2)

        return plsc.parallel_loop(0, NVREG, 1, unroll=UNROLL, carry=carry)(jbody)

    start(0, 0)
    init = (zeros,) * 6

    def outer(i, carry):
        s0 = 2 * i
        start(s0 + 1, 1)
        wait(0)
        carry = chunk_compute(0, carry)

        @pl.when(i < NSTEPS // 2 - 1)
        def _():
            start(s0 + 2, 0)

        wait(1)
        return chunk_compute(1, carry)

    sxr, sxg, sxb, syr, syg, syb = lax.fori_loop(
        0, NSTEPS // 2, outer, init)

    for k, v in enumerate((sxr, sxg, sxb, syr, syg, syb, zeros, zeros)):
        sums_v[pl.ds(k * LANES, LANES)] = v

    pltpu.sync_copy(hx, hx_out.at[pl.ds(img * HSIZE, HSIZE)])
    pltpu.sync_copy(hy, hy_out.at[pl.ds(img * HSIZE, HSIZE)])
    pltpu.sync_copy(sums_v, s_out.at[pl.ds(img * 8 * LANES, 8 * LANES)])


def _sc_hist(xf, yf, wvec):
    mesh = plsc.VectorSubcoreMesh(core_axis_name="c", subcore_axis_name="s",
                                  num_cores=NC, num_subcores=NS)
    f = pl.kernel(
        _sc_hist_body,
        out_type=(
            jax.ShapeDtypeStruct((B * HSIZE,), jnp.float32),
            jax.ShapeDtypeStruct((B * HSIZE,), jnp.float32),
            jax.ShapeDtypeStruct((B * 8 * LANES,), jnp.float32),
        ),
        mesh=mesh,
        scratch_types=[
            pltpu.VMEM((2 * NCH * CHUNK,), jnp.float32),
            pltpu.VMEM((2 * NCH * CHUNK,), jnp.float32),
            pltpu.VMEM((4 * LANES,), jnp.float32),
            pltpu.VMEM((HSIZE,), jnp.float32),
            pltpu.VMEM((HSIZE,), jnp.float32),
            pltpu.VMEM((8 * LANES,), jnp.float32),
            pltpu.SemaphoreType.DMA,
            pltpu.SemaphoreType.DMA,
        ],
        compiler_params=pltpu.CompilerParams(needs_layout_passes=False),
        name="cc_hist_sc",
    )
    return f(xf, yf, wvec)


def _tc_final_body(hx_ref, hy_ref, s_ref, out_ref):
    hx = 2.0 * jnp.sum(hx_ref[...], axis=2)
    hy = 2.0 * jnp.sum(hy_ref[...], axis=2)
    s = jnp.sum(s_ref[...], axis=2)
    m01 = (s * (1.0 / NPIX) + 1.0) * 0.5
    xm = m01[:, 0:3]
    ym = m01[:, 3:6]
    rw = xm[:, 0:1]
    gw = xm[:, 1:2]
    bw = xm[:, 2:3]
    grey_world = jnp.mean(jnp.abs(rw - gw) + jnp.abs(gw - bw) + jnp.abs(bw - rw))
    x_ratio = xm / (jnp.sum(xm, axis=1, keepdims=True) + EPS)
    y_ratio = ym / (jnp.sum(ym, axis=1, keepdims=True) + EPS)
    ratio_loss = jnp.mean(jnp.abs(x_ratio - y_ratio))
    log_x = jnp.log(hx)
    kl_pt = jnp.where(hy > 0,
                      hy * (jnp.log(jnp.where(hy > 0, hy, 1.0)) - log_x), 0.0)
    kl_div = jnp.sum(kl_pt) / B
    out_ref[...] = jnp.full((1, 1), LAMBDA_CC * (grey_world + ratio_loss + kl_div),
                            jnp.float32)


def _tc_final(hx, hy, s):
    return pl.pallas_call(
        _tc_final_body,
        out_shape=jax.ShapeDtypeStruct((1, 1), jnp.float32),
    )(hx, hy, s)


def kernel(x, y, lum_w, bin_edges):
    xf = x.reshape(B * NCH * NPIX)
    yf = y.reshape(B * NCH * NPIX)
    lw = lum_w.reshape(NCH)
    nb = bin_edges.shape[0] - 1
    scale = nb / (bin_edges[-1] - bin_edges[0])
    wrows = 0.5 * scale * lw
    woff = scale * (0.5 * jnp.sum(lw) - bin_edges[0])
    wvec = jnp.concatenate(
        [jnp.broadcast_to(wrows[:, None], (NCH, LANES)),
         jnp.broadcast_to(woff[None, None], (1, LANES))], axis=0).reshape(-1)
    hxf, hyf, sf = _sc_hist(xf, yf, wvec)
    out = _tc_final(hxf.reshape(B, NBINS, LANES),
                    hyf.reshape(B, NBINS, LANES),
                    sf.reshape(B, 8, LANES))
    return out.reshape(())
